# baseline (speedup 1.0000x reference)
import functools
import jax
import jax.numpy as jnp
from jax import lax
from jax.experimental import pallas as pl
from jax.experimental.pallas import tpu as pltpu
from jax.experimental.pallas import tpu_sc as plsc

_NC = 2
_NS = 16
_NW = _NC * _NS
_L = 16


def kernel(x, pos_embed_weight):
    S, B, D = x.shape
    pe = pos_embed_weight[:S]
    rows_per_w = S // _NW
    CS = 8
    n_chunks = rows_per_w // CS
    nvec = D // _L

    mesh = plsc.VectorSubcoreMesh(core_axis_name="c", subcore_axis_name="s")

    @functools.partial(
        pl.kernel,
        mesh=mesh,
        out_type=jax.ShapeDtypeStruct((S, B, D), jnp.float32),
        scratch_types=[
            pltpu.VMEM((CS, B, D), jnp.float32),
            pltpu.VMEM((CS, D), jnp.float32),
            pltpu.SemaphoreType.DMA,
            pltpu.SemaphoreType.DMA,
        ],
    )
    def k(x_hbm, pe_hbm, out_hbm, xb, peb, sem_x, sem_p):
        wid = lax.axis_index("s") * _NC + lax.axis_index("c")
        base = wid * rows_per_w

        def chunk_body(ci, _):
            r0 = base + ci * CS
            cx = pltpu.async_copy(x_hbm.at[pl.ds(r0, CS)], xb, sem_x)
            cp = pltpu.async_copy(pe_hbm.at[pl.ds(r0, CS)], peb, sem_p)
            cx.wait()
            cp.wait()

            def row_body(r, _):
                def vec_body(j, _):
                    sl = pl.ds(j * _L, _L)
                    pev = peb[r, sl]
                    for b in range(B):
                        xb[r, b, sl] = xb[r, b, sl] + pev
                    return 0

                lax.fori_loop(0, nvec, vec_body, 0, unroll=True)
                return 0

            lax.fori_loop(0, CS, row_body, 0)
            pltpu.sync_copy(xb, out_hbm.at[pl.ds(r0, CS)])
            return 0

        lax.fori_loop(0, n_chunks, chunk_body, 0)

    return k(x, pe)


# speedup vs baseline: 1.5637x; 1.5637x over previous
import functools
import jax
import jax.numpy as jnp
from jax import lax
from jax.experimental import pallas as pl
from jax.experimental.pallas import tpu as pltpu
from jax.experimental.pallas import tpu_sc as plsc

_NC = 2
_NS = 16
_NW = _NC * _NS
_L = 16


def kernel(x, pos_embed_weight):
    S, B, D = x.shape
    pe = pos_embed_weight[:S]
    rows_per_w = S // _NW
    CS = 8
    n_chunks = rows_per_w // CS
    nvec = D // _L

    mesh = plsc.VectorSubcoreMesh(core_axis_name="c", subcore_axis_name="s")

    @functools.partial(
        pl.kernel,
        mesh=mesh,
        out_type=jax.ShapeDtypeStruct((S, B, D), jnp.float32),
        scratch_types=[
            pltpu.VMEM((CS, B, D), jnp.float32),
            pltpu.VMEM((CS, B, D), jnp.float32),
            pltpu.VMEM((CS, D), jnp.float32),
            pltpu.VMEM((CS, D), jnp.float32),
            pltpu.SemaphoreType.DMA,
            pltpu.SemaphoreType.DMA,
            pltpu.SemaphoreType.DMA,
            pltpu.SemaphoreType.DMA,
            pltpu.SemaphoreType.DMA,
            pltpu.SemaphoreType.DMA,
        ],
    )
    def k(x_hbm, pe_hbm, out_hbm, xb0, xb1, pb0, pb1,
          six0, six1, sip0, sip1, so0, so1):
        wid = lax.axis_index("s") * _NC + lax.axis_index("c")
        base = wid * rows_per_w
        xbufs = (xb0, xb1)
        pbufs = (pb0, pb1)
        six = (six0, six1)
        sip = (sip0, sip1)
        so = (so0, so1)

        def issue_in(ci):
            p = ci & 1
            r0 = base + ci * CS
            hx = pltpu.async_copy(x_hbm.at[pl.ds(r0, CS)], xbufs[p], six[p])
            hp = pltpu.async_copy(pe_hbm.at[pl.ds(r0, CS)], pbufs[p], sip[p])
            return hx, hp

        def compute(p):
            xb_ = xbufs[p]
            pb_ = pbufs[p]

            def row_body(r, _):
                @plsc.parallel_loop(0, nvec, unroll=8)
                def vec_body(j):
                    sl = pl.ds(j * _L, _L)
                    pev = pb_[r, sl]
                    for b in range(B):
                        xb_[r, b, sl] = xb_[r, b, sl] + pev

                return 0

            lax.fori_loop(0, CS, row_body, 0)

        hin = {0: issue_in(0)}
        hout = {}
        for ci in range(n_chunks):
            p = ci & 1
            if ci + 1 < n_chunks:
                if ci - 1 >= 0:
                    hout[ci - 1].wait()
                hin[ci + 1] = issue_in(ci + 1)
            hx, hp = hin[ci]
            hx.wait()
            hp.wait()
            compute(p)
            r0 = base + ci * CS
            hout[ci] = pltpu.async_copy(xbufs[p], out_hbm.at[pl.ds(r0, CS)], so[p])
        hout[n_chunks - 2].wait()
        hout[n_chunks - 1].wait()

    return k(x, pe)


# speedup vs baseline: 2.9716x; 1.9004x over previous
import jax
import jax.numpy as jnp
from jax.experimental import pallas as pl


def _body(x_ref, pe_ref, o_ref):
    o_ref[...] = x_ref[...] + pe_ref[...][:, None, :]


def kernel(x, pos_embed_weight):
    S, B, D = x.shape
    BS = 512
    return pl.pallas_call(
        _body,
        grid=(S // BS,),
        in_specs=[
            pl.BlockSpec((BS, B, D), lambda i: (i, 0, 0)),
            pl.BlockSpec((BS, D), lambda i: (i, 0)),
        ],
        out_specs=pl.BlockSpec((BS, B, D), lambda i: (i, 0, 0)),
        out_shape=jax.ShapeDtypeStruct((S, B, D), x.dtype),
    )(x, pos_embed_weight[:S])
